# baseline (speedup 1.0000x reference)
import jax
import jax.numpy as jnp
from jax.experimental import pallas as pl
from jax.experimental.pallas import tpu as pltpu

RBLK_A = 256
RBLK = 128
CBLK = 128


def _egat_kernel(ind_smem_ref, att_smem_ref, h_ref, w_ref, att_ref, ai_ref,
                 indc_ref, out_ref,
                 whb_ref, wb_ref, fi_ref, fjt_ref, idsr_ref, sing_ref,
                 acc_ref, s_ref, mtop_ref):
    n, hid = whb_ref.shape
    na = n // RBLK_A
    r = pl.program_id(0)

    @pl.when(r == 0)
    def _prep():
        wb_ref[...] = w_ref[...].astype(jnp.bfloat16)
        ids_c = indc_ref[...]
        idsr_ref[...] = ids_c.reshape(n, 1)
        sing = ((ids_c != pltpu.roll(ids_c, 1, axis=1))
                & (ids_c != pltpu.roll(ids_c, n - 1, axis=1)))
        sing_ref[...] = sing.astype(jnp.int32).reshape(n, 1)

    @pl.when(r < na)
    def _phase_a():
        hb = h_ref[...].astype(jnp.bfloat16)
        wh = jax.lax.dot_general(
            hb, wb_ref[...], (((1,), (1,)), ((), ())),
            preferred_element_type=jnp.float32)
        whb_ref[pl.ds(r * RBLK_A, RBLK_A), :] = wh.astype(jnp.bfloat16)
        ac = att_smem_ref[0, 2 * hid]
        fi_ref[pl.ds(r * RBLK_A, RBLK_A), :] = jnp.dot(
            wh, ai_ref[...], preferred_element_type=jnp.float32) + ac
        aj_row = att_ref[:, hid:2 * hid]
        fjt_ref[:, pl.ds(r * RBLK_A, RBLK_A)] = jax.lax.dot_general(
            aj_row, wh, (((1,), (1,)), ((), ())),
            preferred_element_type=jnp.float32)

    @pl.when(r == na)
    def _mtop():
        m0 = jnp.max(fi_ref[...]) + jnp.max(fjt_ref[...])
        mtop_ref[0, 0] = jnp.maximum(m0, 0.1 * m0)

    @pl.when(r >= na)
    def _phase_b():
        rb = r - na
        m_top = mtop_ref[0, 0]
        fi_r = fi_ref[pl.ds(rb * RBLK, RBLK), :]
        ids_r = idsr_ref[pl.ds(rb * RBLK, RBLK), :]
        r_first = ind_smem_ref[rb * RBLK]
        r_last = ind_smem_ref[rb * RBLK + RBLK - 1]
        acc_ref[...] = jnp.zeros_like(acc_ref)
        s_ref[...] = jnp.zeros_like(s_ref)

        for c in range(n // CBLK):
            @pl.when((ind_smem_ref[c * CBLK] <= r_last)
                     & (ind_smem_ref[c * CBLK + CBLK - 1] >= r_first))
            def _process():
                e0 = fi_r + fjt_ref[:, c * CBLK:(c + 1) * CBLK]
                e = jnp.maximum(e0, 0.1 * e0)
                mask = ids_r == indc_ref[:, c * CBLK:(c + 1) * CBLK]
                p = jnp.where(mask, jnp.exp(e - m_top), 0.0)
                whc = whb_ref[c * CBLK:(c + 1) * CBLK, :]
                acc_ref[...] = acc_ref[...] + jax.lax.dot_general(
                    p.astype(jnp.bfloat16), whc, (((1,), (0,)), ((), ())),
                    preferred_element_type=jnp.float32)
                s_ref[...] = s_ref[...] + jnp.sum(p, axis=1, keepdims=True)

        sing_r = sing_ref[pl.ds(rb * RBLK, RBLK), :]
        out_ref[...] = jnp.where(sing_r == 0, acc_ref[...] / s_ref[...], 0.0)


def kernel(h, ind_id, W, att_w):
    n, hid = h.shape
    indc = ind_id.reshape(1, n)
    ai = att_w[0, :hid].reshape(hid, 1)
    na = n // RBLK_A
    nsteps = na + n // RBLK

    vmem = pl.BlockSpec(memory_space=pltpu.VMEM)
    smem = pl.BlockSpec(memory_space=pltpu.SMEM)
    out = pl.pallas_call(
        _egat_kernel,
        grid=(nsteps,),
        in_specs=[
            smem,
            smem,
            pl.BlockSpec((RBLK_A, hid),
                         lambda r: (jnp.minimum(r, na - 1), 0)),
            vmem,
            vmem,
            vmem,
            vmem,
        ],
        out_specs=pl.BlockSpec((RBLK, hid),
                               lambda r: (jnp.maximum(r - na, 0), 0)),
        out_shape=jax.ShapeDtypeStruct((n, hid), jnp.float32),
        scratch_shapes=[
            pltpu.VMEM((n, hid), jnp.bfloat16),
            pltpu.VMEM((hid, hid), jnp.bfloat16),
            pltpu.VMEM((n, 1), jnp.float32),
            pltpu.VMEM((1, n), jnp.float32),
            pltpu.VMEM((n, 1), jnp.int32),
            pltpu.VMEM((n, 1), jnp.int32),
            pltpu.VMEM((RBLK, hid), jnp.float32),
            pltpu.VMEM((RBLK, 1), jnp.float32),
            pltpu.SMEM((1, 1), jnp.float32),
        ],
    )(ind_id, att_w, h, W, att_w, ai, indc)
    return out


# speedup vs baseline: 1.3908x; 1.3908x over previous
import jax
import jax.numpy as jnp
from jax.experimental import pallas as pl
from jax.experimental.pallas import tpu as pltpu

RBLK = 256
CBLK = 128


def _egat_kernel(ind_smem_ref, att_smem_ref, h_ref, w_ref, att_ref, ai_ref,
                 indc_ref, out_ref,
                 whb_ref, fi_ref, fjt_ref, idsr_ref, sing_ref,
                 acc_ref, s_ref):
    n, hid = h_ref.shape
    hb = h_ref[...].astype(jnp.bfloat16)
    wb = w_ref[...].astype(jnp.bfloat16)
    wh = jax.lax.dot_general(
        hb, wb, (((1,), (1,)), ((), ())), preferred_element_type=jnp.float32)
    whb_ref[...] = wh.astype(jnp.bfloat16)
    ac = att_smem_ref[0, 2 * hid]
    aj_row = att_ref[:, hid:2 * hid]
    fi = jnp.dot(wh, ai_ref[...], preferred_element_type=jnp.float32) + ac
    fi_ref[...] = fi
    fjt = jax.lax.dot_general(
        aj_row, wh, (((1,), (1,)), ((), ())),
        preferred_element_type=jnp.float32)
    fjt_ref[...] = fjt

    ids_c = indc_ref[...]
    idsr_ref[...] = ids_c.reshape(n, 1)
    sing = ((ids_c != pltpu.roll(ids_c, 1, axis=1))
            & (ids_c != pltpu.roll(ids_c, n - 1, axis=1)))
    sing_ref[...] = sing.astype(jnp.int32).reshape(n, 1)

    m0 = jnp.max(fi) + jnp.max(fjt)
    m_top = jnp.maximum(m0, 0.1 * m0)

    def row_body(r, carry_r):
        fi_r = fi_ref[pl.ds(r * RBLK, RBLK), :]
        ids_r = idsr_ref[pl.ds(r * RBLK, RBLK), :]
        r_first = ind_smem_ref[r * RBLK]
        r_last = ind_smem_ref[r * RBLK + RBLK - 1]
        acc_ref[...] = jnp.zeros_like(acc_ref)
        s_ref[...] = jnp.zeros_like(s_ref)

        for c in range(n // CBLK):
            @pl.when((ind_smem_ref[c * CBLK] <= r_last)
                     & (ind_smem_ref[c * CBLK + CBLK - 1] >= r_first))
            def _process():
                e0 = fi_r + fjt_ref[:, c * CBLK:(c + 1) * CBLK]
                e = jnp.maximum(e0, 0.1 * e0)
                mask = ids_r == indc_ref[:, c * CBLK:(c + 1) * CBLK]
                p = jnp.where(mask, jnp.exp(e - m_top), 0.0)
                whc = whb_ref[c * CBLK:(c + 1) * CBLK, :]
                acc_ref[...] = acc_ref[...] + jax.lax.dot_general(
                    p.astype(jnp.bfloat16), whc, (((1,), (0,)), ((), ())),
                    preferred_element_type=jnp.float32)
                s_ref[...] = s_ref[...] + jnp.sum(p, axis=1, keepdims=True)

        sing_r = sing_ref[pl.ds(r * RBLK, RBLK), :]
        out_ref[pl.ds(r * RBLK, RBLK), :] = jnp.where(
            sing_r == 0, acc_ref[...] / s_ref[...], 0.0)
        return carry_r

    jax.lax.fori_loop(0, n // RBLK, row_body, 0)


def kernel(h, ind_id, W, att_w):
    n, hid = h.shape
    indc = ind_id.reshape(1, n)
    ai = att_w[0, :hid].reshape(hid, 1)

    vmem = pl.BlockSpec(memory_space=pltpu.VMEM)
    smem = pl.BlockSpec(memory_space=pltpu.SMEM)
    out = pl.pallas_call(
        _egat_kernel,
        in_specs=[smem, smem, vmem, vmem, vmem, vmem, vmem],
        out_specs=vmem,
        out_shape=jax.ShapeDtypeStruct((n, hid), jnp.float32),
        scratch_shapes=[
            pltpu.VMEM((n, hid), jnp.bfloat16),
            pltpu.VMEM((n, 1), jnp.float32),
            pltpu.VMEM((1, n), jnp.float32),
            pltpu.VMEM((n, 1), jnp.int32),
            pltpu.VMEM((n, 1), jnp.int32),
            pltpu.VMEM((RBLK, hid), jnp.float32),
            pltpu.VMEM((RBLK, 1), jnp.float32),
        ],
    )(ind_id, att_w, h, W, att_w, ai, indc)
    return out


# speedup vs baseline: 1.8518x; 1.3315x over previous
import jax
import jax.numpy as jnp
from jax.experimental import pallas as pl
from jax.experimental.pallas import tpu as pltpu

RBLK = 256
CBLK = 256


def _egat_kernel(ind_smem_ref, att_smem_ref, h_ref, w_ref, att_ref, ai_ref,
                 indc_ref, out_ref,
                 whb_ref, fi_ref, fjt_ref, idsr_ref, sing_ref,
                 acc_ref, s_ref, mtop_ref):
    n, hid = h_ref.shape
    r = pl.program_id(0)

    @pl.when(r == 0)
    def _prologue():
        hb = h_ref[...].astype(jnp.bfloat16)
        wb = w_ref[...].astype(jnp.bfloat16)
        wh = jax.lax.dot_general(
            hb, wb, (((1,), (1,)), ((), ())),
            preferred_element_type=jnp.float32)
        whb_ref[...] = wh.astype(jnp.bfloat16)
        ac = att_smem_ref[0, 2 * hid]
        aj_row = att_ref[:, hid:2 * hid]
        fi = jnp.dot(wh, ai_ref[...],
                     preferred_element_type=jnp.float32) + ac
        fi_ref[...] = fi
        fjt = jax.lax.dot_general(
            aj_row, wh, (((1,), (1,)), ((), ())),
            preferred_element_type=jnp.float32)
        fjt_ref[...] = fjt

        ids_c = indc_ref[...]
        idsr_ref[...] = ids_c.reshape(n, 1)
        sing = ((ids_c != pltpu.roll(ids_c, 1, axis=1))
                & (ids_c != pltpu.roll(ids_c, n - 1, axis=1)))
        sing_ref[...] = sing.astype(jnp.int32).reshape(n, 1)

        m0 = jnp.max(fi) + jnp.max(fjt)
        mtop_ref[0, 0] = jnp.maximum(m0, 0.1 * m0)

    m_top = mtop_ref[0, 0]
    fi_r = fi_ref[pl.ds(r * RBLK, RBLK), :]
    ids_r = idsr_ref[pl.ds(r * RBLK, RBLK), :]
    r_first = ind_smem_ref[r * RBLK]
    r_last = ind_smem_ref[r * RBLK + RBLK - 1]
    acc_ref[...] = jnp.zeros_like(acc_ref)
    s_ref[...] = jnp.zeros_like(s_ref)

    for c in range(n // CBLK):
        @pl.when((ind_smem_ref[c * CBLK] <= r_last)
                 & (ind_smem_ref[c * CBLK + CBLK - 1] >= r_first))
        def _process():
            e0 = fi_r + fjt_ref[:, c * CBLK:(c + 1) * CBLK]
            e = jnp.maximum(e0, 0.1 * e0)
            mask = ids_r == indc_ref[:, c * CBLK:(c + 1) * CBLK]
            p = jnp.where(mask, jnp.exp(e - m_top), 0.0)
            whc = whb_ref[c * CBLK:(c + 1) * CBLK, :]
            acc_ref[...] = acc_ref[...] + jax.lax.dot_general(
                p.astype(jnp.bfloat16), whc, (((1,), (0,)), ((), ())),
                preferred_element_type=jnp.float32)
            s_ref[...] = s_ref[...] + jnp.sum(p, axis=1, keepdims=True)

    sing_r = sing_ref[pl.ds(r * RBLK, RBLK), :]
    out_ref[...] = jnp.where(
        sing_r == 0, acc_ref[...] / s_ref[...], 0.0)


def kernel(h, ind_id, W, att_w):
    n, hid = h.shape
    indc = ind_id.reshape(1, n)
    ai = att_w[0, :hid].reshape(hid, 1)

    vmem = pl.BlockSpec(memory_space=pltpu.VMEM)
    smem = pl.BlockSpec(memory_space=pltpu.SMEM)
    out = pl.pallas_call(
        _egat_kernel,
        grid=(n // RBLK,),
        in_specs=[smem, smem, vmem, vmem, vmem, vmem, vmem],
        out_specs=pl.BlockSpec((RBLK, hid), lambda r: (r, 0)),
        out_shape=jax.ShapeDtypeStruct((n, hid), jnp.float32),
        scratch_shapes=[
            pltpu.VMEM((n, hid), jnp.bfloat16),
            pltpu.VMEM((n, 1), jnp.float32),
            pltpu.VMEM((1, n), jnp.float32),
            pltpu.VMEM((n, 1), jnp.int32),
            pltpu.VMEM((n, 1), jnp.int32),
            pltpu.VMEM((RBLK, hid), jnp.float32),
            pltpu.VMEM((RBLK, 1), jnp.float32),
            pltpu.SMEM((1, 1), jnp.float32),
        ],
    )(ind_id, att_w, h, W, att_w, ai, indc)
    return out
